# Initial kernel scaffold; baseline (speedup 1.0000x reference)
import jax
import jax.numpy as jnp
from jax.experimental import pallas as pl


def kernel(nf, ef, edge_index, W_ne, b_ne, W_ee, b_ee, eW1, eb1, eW2, eb2, nW1, nb1, nW2, nb2, pW, pb):
    raise NotImplementedError("write your pallas kernel here")


# speedup vs baseline: 2.2621x; 2.2621x over previous
import functools

import jax
import jax.numpy as jnp
from jax import lax
from jax.experimental import pallas as pl
from jax.experimental.pallas import tpu as pltpu
from jax.experimental.pallas import tpu_sc as plsc

F32 = jnp.float32
_NC, _NS = 2, 16
_NW = _NC * _NS
_LANES = 16
_CH = 128

_PREC = lax.Precision.HIGHEST


def _dot(a, b):
    return jnp.dot(a, b, preferred_element_type=F32, precision=_PREC)


def _mm_bias_body(x_ref, w_ref, b_ref, o_ref):
    o_ref[...] = _dot(x_ref[...], w_ref[...]) + b_ref[...]


def _mm_bias(x, w, b, bm):
    m, k = x.shape
    n = w.shape[1]
    return pl.pallas_call(
        _mm_bias_body,
        grid=(m // bm,),
        in_specs=[
            pl.BlockSpec((bm, k), lambda i: (i, 0)),
            pl.BlockSpec((k, n), lambda i: (0, 0)),
            pl.BlockSpec((1, n), lambda i: (0, 0)),
        ],
        out_specs=pl.BlockSpec((bm, n), lambda i: (i, 0)),
        out_shape=jax.ShapeDtypeStruct((m, n), F32),
    )(x, w, b.reshape(1, n))


def _pq_body(x_ref, a_ref, b_ref, p_ref, q_ref):
    x = x_ref[...]
    p_ref[...] = _dot(x, a_ref[...])
    q_ref[...] = _dot(x, b_ref[...])


def _pq_project(unf, A, B, bm):
    m, k = unf.shape
    n = A.shape[1]
    return pl.pallas_call(
        _pq_body,
        grid=(m // bm,),
        in_specs=[
            pl.BlockSpec((bm, k), lambda i: (i, 0)),
            pl.BlockSpec((k, n), lambda i: (0, 0)),
            pl.BlockSpec((k, n), lambda i: (0, 0)),
        ],
        out_specs=[
            pl.BlockSpec((bm, n), lambda i: (i, 0)),
            pl.BlockSpec((bm, n), lambda i: (i, 0)),
        ],
        out_shape=[
            jax.ShapeDtypeStruct((m, n), F32),
            jax.ShapeDtypeStruct((m, n), F32),
        ],
    )(unf, A, B)


def _edge0_body(u_ref, g_ref, c_ref, w2_ref, b1_ref, b2_ref, o_ref):
    h = jnp.maximum(_dot(u_ref[...], c_ref[...]) + g_ref[...] + b1_ref[...], 0.0)
    o_ref[...] = _dot(h, w2_ref[...]) + b2_ref[...]


def _edge_res_body(u_ref, r_ref, g_ref, c_ref, w2_ref, b1_ref, b2_ref, o_ref):
    u = u_ref[...] + r_ref[...]
    h = jnp.maximum(_dot(u, c_ref[...]) + g_ref[...] + b1_ref[...], 0.0)
    o_ref[...] = _dot(h, w2_ref[...]) + b2_ref[...]


def _edge_mlp(u, res, g, C, W2, b1, b2, bm):
    m, n = u.shape
    spec = pl.BlockSpec((bm, n), lambda i: (i, 0))
    wspec = pl.BlockSpec((n, n), lambda i: (0, 0))
    bspec = pl.BlockSpec((1, n), lambda i: (0, 0))
    if res is None:
        body = _edge0_body
        args = (u, g, C, W2, b1.reshape(1, n), b2.reshape(1, n))
        in_specs = [spec, spec, wspec, wspec, bspec, bspec]
    else:
        body = _edge_res_body
        args = (u, res, g, C, W2, b1.reshape(1, n), b2.reshape(1, n))
        in_specs = [spec, spec, spec, wspec, wspec, bspec, bspec]
    return pl.pallas_call(
        body,
        grid=(m // bm,),
        in_specs=in_specs,
        out_specs=spec,
        out_shape=jax.ShapeDtypeStruct((m, n), F32),
    )(*args)


def _node_body(x_ref, a0_ref, a1_ref, u_ref, v_ref, w2_ref, b1_ref, b2_ref,
               res_ref, o_ref):
    x = x_ref[...]
    a = a0_ref[0] + a1_ref[0]
    h = jnp.maximum(_dot(x, u_ref[...]) + _dot(a, v_ref[...]) + b1_ref[...], 0.0)
    o_ref[...] = _dot(h, w2_ref[...]) + b2_ref[...] + res_ref[...]


def _node_mlp(x, agg, U, V, W2, b1, b2, res, bm):
    m, n = x.shape
    spec = pl.BlockSpec((bm, n), lambda i: (i, 0))
    wspec = pl.BlockSpec((n, n), lambda i: (0, 0))
    bspec = pl.BlockSpec((1, n), lambda i: (0, 0))
    aspec = pl.BlockSpec((1, bm, n), lambda i: (0, i, 0))
    return pl.pallas_call(
        _node_body,
        grid=(m // bm,),
        in_specs=[spec, aspec, aspec, wspec, wspec, wspec, bspec, bspec, spec],
        out_specs=spec,
        out_shape=jax.ShapeDtypeStruct((m, n), F32),
    )(x, agg[0:1], agg[1:2], U, V, W2, b1.reshape(1, n), b2.reshape(1, n), res)


def _policy_body(x_ref, w_ref, b_ref, o_ref):
    z = _dot(x_ref[...], w_ref[...]) + b_ref[...]
    o_ref[...] = jnp.clip(jax.nn.sigmoid(z), 0.0, 1.0)


def _policy(x, w, b, bm):
    m, k = x.shape
    return pl.pallas_call(
        _policy_body,
        grid=(m // bm,),
        in_specs=[
            pl.BlockSpec((bm, k), lambda i: (i, 0)),
            pl.BlockSpec((k, 1), lambda i: (0, 0)),
            pl.BlockSpec((1, 1), lambda i: (0, 0)),
        ],
        out_specs=pl.BlockSpec((bm, 1), lambda i: (i, 0)),
        out_shape=jax.ShapeDtypeStruct((m, 1), F32),
    )(x, w, b.reshape(1, 1))


def _sc_mesh():
    return plsc.VectorSubcoreMesh(core_axis_name="c", subcore_axis_name="s",
                                  num_cores=_NC, num_subcores=_NS)


def _sc_gather_call(P, Q, sidx, didx):
    e_total, n = P.shape[0], P.shape[1]
    n_ch = sidx.shape[0] // _CH
    nj_max = -(-n_ch // _NW)

    @functools.partial(
        pl.kernel,
        out_type=jax.ShapeDtypeStruct((sidx.shape[0], n), F32),
        mesh=_sc_mesh(),
        scratch_types=[
            pltpu.VMEM((_CH,), jnp.int32),
            pltpu.VMEM((_CH,), jnp.int32),
            pltpu.VMEM((_CH, n), F32),
            pltpu.VMEM((_CH, n), F32),
            pltpu.SemaphoreType.DMA,
            pltpu.SemaphoreType.DMA,
        ],
    )
    def k(p_hbm, q_hbm, si_hbm, di_hbm, out_hbm, si_v, di_v, pg_v, qg_v,
          sem1, sem2):
        wid = lax.axis_index("s") * _NC + lax.axis_index("c")

        def body(j, carry):
            c = wid + j * _NW

            @pl.when(c < n_ch)
            def _():
                off = c * _CH
                pltpu.sync_copy(si_hbm.at[pl.ds(off, _CH)], si_v)
                pltpu.sync_copy(di_hbm.at[pl.ds(off, _CH)], di_v)
                cp1 = pltpu.async_copy(p_hbm.at[si_v], pg_v, sem1)
                cp2 = pltpu.async_copy(q_hbm.at[di_v], qg_v, sem2)
                cp1.wait()
                cp2.wait()

                def add_row(i, carry2):
                    for t in range(n // _LANES):
                        s = pl.ds(t * _LANES, _LANES)
                        pg_v[i, s] = pg_v[i, s] + qg_v[i, s]
                    return carry2

                lax.fori_loop(0, _CH, add_row, 0)
                pltpu.sync_copy(pg_v, out_hbm.at[pl.ds(off, _CH)])

            return carry

        lax.fori_loop(0, nj_max, body, 0)

    return k(P, Q, sidx, didx)


def _sc_scatter_call(e_feat, didx, n_nodes):
    n = e_feat.shape[1]
    n_ch = didx.shape[0] // _CH
    nj_max = -(-n_ch // _NW)
    zrows = 128
    rows_per_tile = -(-n_nodes // (_NS * zrows)) * zrows
    n_pad = rows_per_tile * _NS

    @functools.partial(
        pl.kernel,
        out_type=jax.ShapeDtypeStruct((_NC, n_pad, n), F32),
        mesh=_sc_mesh(),
        scratch_types=[
            pltpu.VMEM((nj_max, _CH), jnp.int32),
            pltpu.VMEM((_CH, n), F32),
            pltpu.VMEM((zrows, n), F32),
            pltpu.VMEM_SHARED((n_pad, n), F32),
            pltpu.SemaphoreType.DMA,
        ],
    )
    def k(e_hbm, di_hbm, out_hbm, idx_v, rows_v, zb_v, agg_sh, sem):
        cid = lax.axis_index("c")
        sid = lax.axis_index("s")
        wid = sid * _NC + cid

        def zrow(i, carry):
            for t in range(n // _LANES):
                zb_v[i, pl.ds(t * _LANES, _LANES)] = jnp.zeros((_LANES,), F32)
            return carry

        lax.fori_loop(0, zrows, zrow, 0)
        for t in range(rows_per_tile // zrows):
            pltpu.sync_copy(
                zb_v, agg_sh.at[pl.ds(sid * rows_per_tile + t * zrows, zrows)])
        plsc.subcore_barrier()

        def body(j, carry):
            c = wid + j * _NW

            @pl.when(c < n_ch)
            def _():
                off = c * _CH
                pltpu.sync_copy(di_hbm.at[pl.ds(off, _CH)], idx_v.at[j])
                pltpu.sync_copy(e_hbm.at[pl.ds(off, _CH)], rows_v)
                pltpu.sync_copy(rows_v, agg_sh.at[idx_v.at[j]], add=True)

            return carry

        lax.fori_loop(0, nj_max, body, 0)
        plsc.subcore_barrier()

        for t in range(rows_per_tile // zrows):
            sl = pl.ds(sid * rows_per_tile + t * zrows, zrows)
            pltpu.sync_copy(agg_sh.at[sl], zb_v)
            pltpu.sync_copy(zb_v, out_hbm.at[cid, sl])

    return k(e_feat, didx)


def kernel(nf, ef, edge_index, W_ne, b_ne, W_ee, b_ee, eW1, eb1, eW2, eb2,
           nW1, nb1, nW2, nb2, pW, pb):
    n_nodes, latent = nf.shape[0], W_ne.shape[1]
    n_layers = eW1.shape[0]
    bn = 2000
    be = 2560

    sidx = edge_index[0].astype(jnp.int32)
    didx = edge_index[1].astype(jnp.int32)

    unf0 = _mm_bias(nf, W_ne, b_ne, bn)
    uef0 = _mm_bias(ef, W_ee, b_ee, be)

    unf = unf0
    e_prev = None
    for l in range(n_layers):
        A = eW1[l, :latent, :]
        B = eW1[l, latent:2 * latent, :]
        C = eW1[l, 2 * latent:, :]
        P, Q = _pq_project(unf, A, B, bn)
        G = _sc_gather_call(P, Q, sidx, didx)
        if e_prev is None:
            e_new = _edge_mlp(uef0, None, G, C, eW2[l], eb1[l], eb2[l], be)
        else:
            e_new = _edge_mlp(e_prev, uef0, G, C, eW2[l], eb1[l], eb2[l], be)
        agg = _sc_scatter_call(e_new, didx, n_nodes)[:, :n_nodes, :]
        U = nW1[l, :latent, :]
        V = nW1[l, latent:, :]
        unf = _node_mlp(unf, agg, U, V, nW2[l], nb1[l], nb2[l], unf0, bn)
        e_prev = e_new

    return _policy(unf, pW, pb, bn)


# speedup vs baseline: 2.5825x; 1.1416x over previous
import functools

import jax
import jax.numpy as jnp
from jax import lax
from jax.experimental import pallas as pl
from jax.experimental.pallas import tpu as pltpu
from jax.experimental.pallas import tpu_sc as plsc

F32 = jnp.float32
_NC, _NS = 2, 16
_NW = _NC * _NS
_LANES = 16
_CH = 128

_PREC = lax.Precision.HIGHEST


def _dot(a, b):
    return jnp.dot(a, b, preferred_element_type=F32, precision=_PREC)


def _mm_bias_body(x_ref, w_ref, b_ref, o_ref):
    o_ref[...] = _dot(x_ref[...], w_ref[...]) + b_ref[...]


def _mm_bias(x, w, b, bm):
    m, k = x.shape
    n = w.shape[1]
    return pl.pallas_call(
        _mm_bias_body,
        grid=(m // bm,),
        in_specs=[
            pl.BlockSpec((bm, k), lambda i: (i, 0)),
            pl.BlockSpec((k, n), lambda i: (0, 0)),
            pl.BlockSpec((1, n), lambda i: (0, 0)),
        ],
        out_specs=pl.BlockSpec((bm, n), lambda i: (i, 0)),
        out_shape=jax.ShapeDtypeStruct((m, n), F32),
    )(x, w, b.reshape(1, n))


def _pq_body(x_ref, a_ref, b_ref, p_ref, q_ref):
    x = x_ref[...]
    p_ref[...] = _dot(x, a_ref[...])
    q_ref[...] = _dot(x, b_ref[...])


def _pq_project(unf, A, B, bm):
    m, k = unf.shape
    n = A.shape[1]
    return pl.pallas_call(
        _pq_body,
        grid=(m // bm,),
        in_specs=[
            pl.BlockSpec((bm, k), lambda i: (i, 0)),
            pl.BlockSpec((k, n), lambda i: (0, 0)),
            pl.BlockSpec((k, n), lambda i: (0, 0)),
        ],
        out_specs=[
            pl.BlockSpec((bm, n), lambda i: (i, 0)),
            pl.BlockSpec((bm, n), lambda i: (i, 0)),
        ],
        out_shape=[
            jax.ShapeDtypeStruct((m, n), F32),
            jax.ShapeDtypeStruct((m, n), F32),
        ],
    )(unf, A, B)


def _edge0_body(u_ref, g_ref, c_ref, w2_ref, b1_ref, b2_ref, o_ref, *, nreal):
    @pl.when(pl.program_id(0) < nreal)
    def _():
        h = jnp.maximum(
            _dot(u_ref[...], c_ref[...]) + g_ref[...] + b1_ref[...], 0.0)
        o_ref[...] = _dot(h, w2_ref[...]) + b2_ref[...]

    @pl.when(pl.program_id(0) >= nreal)
    def _():
        o_ref[...] = jnp.zeros_like(o_ref)


def _edge_res_body(u_ref, r_ref, g_ref, c_ref, w2_ref, b1_ref, b2_ref, o_ref,
                   *, nreal):
    @pl.when(pl.program_id(0) < nreal)
    def _():
        u = u_ref[...] + r_ref[...]
        h = jnp.maximum(_dot(u, c_ref[...]) + g_ref[...] + b1_ref[...], 0.0)
        o_ref[...] = _dot(h, w2_ref[...]) + b2_ref[...]

    @pl.when(pl.program_id(0) >= nreal)
    def _():
        o_ref[...] = jnp.zeros_like(o_ref)


def _edge_mlp(u, res, g, C, W2, b1, b2, bm, nreal):
    n = u.shape[1]
    m_pad = g.shape[0]

    def spec(arr):
        nb = arr.shape[0] // bm
        if nb == m_pad // bm:
            return pl.BlockSpec((bm, n), lambda i: (i, 0))
        return pl.BlockSpec((bm, n),
                            lambda i, nb=nb: (jnp.minimum(i, nb - 1), 0))

    wspec = pl.BlockSpec((n, n), lambda i: (0, 0))
    bspec = pl.BlockSpec((1, n), lambda i: (0, 0))
    if res is None:
        body = functools.partial(_edge0_body, nreal=nreal)
        args = (u, g, C, W2, b1.reshape(1, n), b2.reshape(1, n))
        in_specs = [spec(u), spec(g), wspec, wspec, bspec, bspec]
    else:
        body = functools.partial(_edge_res_body, nreal=nreal)
        args = (u, res, g, C, W2, b1.reshape(1, n), b2.reshape(1, n))
        in_specs = [spec(u), spec(res), spec(g), wspec, wspec, bspec, bspec]
    return pl.pallas_call(
        body,
        grid=(m_pad // bm,),
        in_specs=in_specs,
        out_specs=spec(g),
        out_shape=jax.ShapeDtypeStruct((m_pad, n), F32),
    )(*args)


def _node_body(x_ref, a0_ref, a1_ref, u_ref, v_ref, w2_ref, b1_ref, b2_ref,
               res_ref, o_ref):
    x = x_ref[...]
    a = a0_ref[0] + a1_ref[0]
    h = jnp.maximum(_dot(x, u_ref[...]) + _dot(a, v_ref[...]) + b1_ref[...], 0.0)
    o_ref[...] = _dot(h, w2_ref[...]) + b2_ref[...] + res_ref[...]


def _node_mlp(x, agg, U, V, W2, b1, b2, res, bm):
    m, n = x.shape
    spec = pl.BlockSpec((bm, n), lambda i: (i, 0))
    wspec = pl.BlockSpec((n, n), lambda i: (0, 0))
    bspec = pl.BlockSpec((1, n), lambda i: (0, 0))
    aspec = pl.BlockSpec((1, bm, n), lambda i: (0, i, 0))
    return pl.pallas_call(
        _node_body,
        grid=(m // bm,),
        in_specs=[spec, aspec, aspec, wspec, wspec, wspec, bspec, bspec, spec],
        out_specs=spec,
        out_shape=jax.ShapeDtypeStruct((m, n), F32),
    )(x, agg[0:1], agg[1:2], U, V, W2, b1.reshape(1, n), b2.reshape(1, n), res)


def _policy_body(x_ref, w_ref, b_ref, o_ref):
    z = _dot(x_ref[...], w_ref[...]) + b_ref[...]
    o_ref[...] = jnp.clip(jax.nn.sigmoid(z), 0.0, 1.0)


def _policy(x, w, b, bm):
    m, k = x.shape
    return pl.pallas_call(
        _policy_body,
        grid=(m // bm,),
        in_specs=[
            pl.BlockSpec((bm, k), lambda i: (i, 0)),
            pl.BlockSpec((k, 1), lambda i: (0, 0)),
            pl.BlockSpec((1, 1), lambda i: (0, 0)),
        ],
        out_specs=pl.BlockSpec((bm, 1), lambda i: (i, 0)),
        out_shape=jax.ShapeDtypeStruct((m, 1), F32),
    )(x, w, b.reshape(1, 1))


def _sc_mesh():
    return plsc.VectorSubcoreMesh(core_axis_name="c", subcore_axis_name="s",
                                  num_cores=_NC, num_subcores=_NS)


def _sc_gather_call(P, Q, si2, di2):
    n = P.shape[1]
    n_ch = si2.shape[0]
    q = n_ch // _NW
    npairs = q // 2

    @functools.partial(
        pl.kernel,
        out_type=jax.ShapeDtypeStruct((n_ch * _CH, n), F32),
        mesh=_sc_mesh(),
        scratch_types=[
            pltpu.VMEM((q, _CH), jnp.int32),
            pltpu.VMEM((q, _CH), jnp.int32),
            pltpu.VMEM((_CH, n), F32),
            pltpu.VMEM((_CH, n), F32),
            pltpu.VMEM((_CH, n), F32),
            pltpu.VMEM((_CH, n), F32),
            pltpu.SemaphoreType.DMA,
            pltpu.SemaphoreType.DMA,
            pltpu.SemaphoreType.DMA,
            pltpu.SemaphoreType.DMA,
        ],
    )
    def k(p_hbm, q_hbm, si_hbm, di_hbm, out_hbm, si_v, di_v,
          pg0, qg0, pg1, qg1, sg0, sg1, sw0, sw1):
        wid = lax.axis_index("s") * _NC + lax.axis_index("c")
        start = wid * q
        pltpu.sync_copy(si_hbm.at[pl.ds(start, q)], si_v)
        pltpu.sync_copy(di_hbm.at[pl.ds(start, q)], di_v)

        def add_rows(pg, qg):
            def add_row(i, carry):
                for t in range(n // _LANES):
                    s = pl.ds(t * _LANES, _LANES)
                    pg[i, s] = pg[i, s] + qg[i, s]
                return carry

            lax.fori_loop(0, _CH, add_row, 0)

        def pair(j2, carry):
            j0 = j2 * 2
            j1 = j0 + 1
            cp0 = pltpu.async_copy(p_hbm.at[si_v.at[j0]], pg0, sg0)
            cq0 = pltpu.async_copy(q_hbm.at[di_v.at[j0]], qg0, sg0)
            cp1 = pltpu.async_copy(p_hbm.at[si_v.at[j1]], pg1, sg1)
            cq1 = pltpu.async_copy(q_hbm.at[di_v.at[j1]], qg1, sg1)
            cp0.wait()
            cq0.wait()
            add_rows(pg0, qg0)
            w0 = pltpu.async_copy(
                pg0, out_hbm.at[pl.ds((start + j0) * _CH, _CH)], sw0)
            cp1.wait()
            cq1.wait()
            add_rows(pg1, qg1)
            w1 = pltpu.async_copy(
                pg1, out_hbm.at[pl.ds((start + j1) * _CH, _CH)], sw1)
            w0.wait()
            w1.wait()
            return carry

        lax.fori_loop(0, npairs, pair, 0)

    return k(P, Q, si2, di2)


def _sc_scatter_call(e_feat, di2, n_nodes):
    n = e_feat.shape[1]
    n_ch = di2.shape[0]
    q = n_ch // _NW
    npairs = q // 2
    zrows = 128
    rows_per_tile = -(-n_nodes // (_NS * zrows)) * zrows
    n_pad = rows_per_tile * _NS

    @functools.partial(
        pl.kernel,
        out_type=jax.ShapeDtypeStruct((_NC, n_pad, n), F32),
        mesh=_sc_mesh(),
        scratch_types=[
            pltpu.VMEM((q, _CH), jnp.int32),
            pltpu.VMEM((_CH, n), F32),
            pltpu.VMEM((_CH, n), F32),
            pltpu.VMEM_SHARED((n_pad, n), F32),
            pltpu.SemaphoreType.DMA,
            pltpu.SemaphoreType.DMA,
            pltpu.SemaphoreType.DMA,
            pltpu.SemaphoreType.DMA,
        ],
    )
    def k(e_hbm, di_hbm, out_hbm, idx_v, rows0, rows1, agg_sh,
          sl0, sl1, ss0, ss1):
        cid = lax.axis_index("c")
        sid = lax.axis_index("s")
        wid = sid * _NC + cid
        start = wid * q

        def zrow(i, carry):
            for t in range(n // _LANES):
                rows0[i, pl.ds(t * _LANES, _LANES)] = jnp.zeros((_LANES,), F32)
            return carry

        lax.fori_loop(0, zrows, zrow, 0)
        for t in range(rows_per_tile // zrows):
            pltpu.sync_copy(
                rows0,
                agg_sh.at[pl.ds(sid * rows_per_tile + t * zrows, zrows)])
        pltpu.sync_copy(di_hbm.at[pl.ds(start, q)], idx_v)
        plsc.subcore_barrier()

        def pair(j2, carry):
            j0 = j2 * 2
            j1 = j0 + 1
            l0 = pltpu.async_copy(
                e_hbm.at[pl.ds((start + j0) * _CH, _CH)], rows0, sl0)
            l1 = pltpu.async_copy(
                e_hbm.at[pl.ds((start + j1) * _CH, _CH)], rows1, sl1)
            l0.wait()
            s0 = pltpu.async_copy(rows0, agg_sh.at[idx_v.at[j0]], ss0,
                                  add=True)
            l1.wait()
            s1 = pltpu.async_copy(rows1, agg_sh.at[idx_v.at[j1]], ss1,
                                  add=True)
            s0.wait()
            s1.wait()
            return carry

        lax.fori_loop(0, npairs, pair, 0)
        plsc.subcore_barrier()

        for t in range(rows_per_tile // zrows):
            sl = pl.ds(sid * rows_per_tile + t * zrows, zrows)
            pltpu.sync_copy(agg_sh.at[sl], rows0)
            pltpu.sync_copy(rows0, out_hbm.at[cid, sl])

    return k(e_feat, di2)


def kernel(nf, ef, edge_index, W_ne, b_ne, W_ee, b_ee, eW1, eb1, eW2, eb2,
           nW1, nb1, nW2, nb2, pW, pb):
    n_nodes, latent = nf.shape[0], W_ne.shape[1]
    n_layers = eW1.shape[0]
    bn = 2000
    be = 2560

    sidx = edge_index[0].astype(jnp.int32)
    didx = edge_index[1].astype(jnp.int32)

    e_total = sidx.shape[0]
    n_ch = -(-(e_total // _CH) // (2 * _NW)) * (2 * _NW)
    pad = n_ch * _CH - e_total
    pad_idx = jnp.arange(pad, dtype=jnp.int32) % n_nodes
    si2 = jnp.concatenate([sidx, pad_idx]).reshape(n_ch, _CH)
    di2 = jnp.concatenate([didx, pad_idx]).reshape(n_ch, _CH)

    unf0 = _mm_bias(nf, W_ne, b_ne, bn)
    uef0 = _mm_bias(ef, W_ee, b_ee, be)

    unf = unf0
    e_prev = None
    for l in range(n_layers):
        A = eW1[l, :latent, :]
        B = eW1[l, latent:2 * latent, :]
        C = eW1[l, 2 * latent:, :]
        P, Q = _pq_project(unf, A, B, bn)
        G = _sc_gather_call(P, Q, si2, di2)
        nreal = e_total // be
        if e_prev is None:
            e_new = _edge_mlp(uef0, None, G, C, eW2[l], eb1[l], eb2[l], be,
                              nreal)
        else:
            e_new = _edge_mlp(e_prev, uef0, G, C, eW2[l], eb1[l], eb2[l], be,
                              nreal)
        agg = _sc_scatter_call(e_new, di2, n_nodes)[:, :n_nodes, :]
        U = nW1[l, :latent, :]
        V = nW1[l, latent:, :]
        unf = _node_mlp(unf, agg, U, V, nW2[l], nb1[l], nb2[l], unf0, bn)
        e_prev = e_new

    return _policy(unf, pW, pb, bn)


# speedup vs baseline: 3.9558x; 1.5318x over previous
import functools

import jax
import jax.numpy as jnp
from jax import lax
from jax.experimental import pallas as pl
from jax.experimental.pallas import tpu as pltpu
from jax.experimental.pallas import tpu_sc as plsc

F32 = jnp.float32
_NC, _NS = 2, 16
_NW = _NC * _NS
_LANES = 16
_CH = 128

def _dot(a, b):
    return jnp.dot(a.astype(jnp.bfloat16), b.astype(jnp.bfloat16),
                   preferred_element_type=F32)


def _mm_bias_body(x_ref, w_ref, b_ref, o_ref):
    o_ref[...] = _dot(x_ref[...], w_ref[...]) + b_ref[...]


def _mm_bias(x, w, b, bm):
    m, k = x.shape
    n = w.shape[1]
    return pl.pallas_call(
        _mm_bias_body,
        grid=(m // bm,),
        in_specs=[
            pl.BlockSpec((bm, k), lambda i: (i, 0)),
            pl.BlockSpec((k, n), lambda i: (0, 0)),
            pl.BlockSpec((1, n), lambda i: (0, 0)),
        ],
        out_specs=pl.BlockSpec((bm, n), lambda i: (i, 0)),
        out_shape=jax.ShapeDtypeStruct((m, n), F32),
    )(x, w, b.reshape(1, n))


def _pq_body(x_ref, a_ref, b_ref, p_ref, q_ref):
    x = x_ref[...]
    p_ref[...] = _dot(x, a_ref[...])
    q_ref[...] = _dot(x, b_ref[...])


def _pq_project(unf, A, B, bm):
    m, k = unf.shape
    n = A.shape[1]
    return pl.pallas_call(
        _pq_body,
        grid=(m // bm,),
        in_specs=[
            pl.BlockSpec((bm, k), lambda i: (i, 0)),
            pl.BlockSpec((k, n), lambda i: (0, 0)),
            pl.BlockSpec((k, n), lambda i: (0, 0)),
        ],
        out_specs=[
            pl.BlockSpec((bm, n), lambda i: (i, 0)),
            pl.BlockSpec((bm, n), lambda i: (i, 0)),
        ],
        out_shape=[
            jax.ShapeDtypeStruct((m, n), F32),
            jax.ShapeDtypeStruct((m, n), F32),
        ],
    )(unf, A, B)


def _edge0_body(u_ref, g_ref, c_ref, w2_ref, b1_ref, b2_ref, o_ref, *, nreal):
    @pl.when(pl.program_id(0) < nreal)
    def _():
        h = jnp.maximum(
            _dot(u_ref[...], c_ref[...]) + g_ref[...] + b1_ref[...], 0.0)
        o_ref[...] = _dot(h, w2_ref[...]) + b2_ref[...]

    @pl.when(pl.program_id(0) >= nreal)
    def _():
        o_ref[...] = jnp.zeros_like(o_ref)


def _edge_res_body(u_ref, r_ref, g_ref, c_ref, w2_ref, b1_ref, b2_ref, o_ref,
                   *, nreal):
    @pl.when(pl.program_id(0) < nreal)
    def _():
        u = u_ref[...] + r_ref[...]
        h = jnp.maximum(_dot(u, c_ref[...]) + g_ref[...] + b1_ref[...], 0.0)
        o_ref[...] = _dot(h, w2_ref[...]) + b2_ref[...]

    @pl.when(pl.program_id(0) >= nreal)
    def _():
        o_ref[...] = jnp.zeros_like(o_ref)


def _edge_mlp(u, res, g, C, W2, b1, b2, bm, nreal):
    n = u.shape[1]
    m_pad = g.shape[0]

    def spec(arr):
        nb = arr.shape[0] // bm
        if nb == m_pad // bm:
            return pl.BlockSpec((bm, n), lambda i: (i, 0))
        return pl.BlockSpec((bm, n),
                            lambda i, nb=nb: (jnp.minimum(i, nb - 1), 0))

    wspec = pl.BlockSpec((n, n), lambda i: (0, 0))
    bspec = pl.BlockSpec((1, n), lambda i: (0, 0))
    if res is None:
        body = functools.partial(_edge0_body, nreal=nreal)
        args = (u, g, C, W2, b1.reshape(1, n), b2.reshape(1, n))
        in_specs = [spec(u), spec(g), wspec, wspec, bspec, bspec]
    else:
        body = functools.partial(_edge_res_body, nreal=nreal)
        args = (u, res, g, C, W2, b1.reshape(1, n), b2.reshape(1, n))
        in_specs = [spec(u), spec(res), spec(g), wspec, wspec, bspec, bspec]
    return pl.pallas_call(
        body,
        grid=(m_pad // bm,),
        in_specs=in_specs,
        out_specs=spec(g),
        out_shape=jax.ShapeDtypeStruct((m_pad, n), F32),
    )(*args)


def _node_body(x_ref, a0_ref, a1_ref, u_ref, v_ref, w2_ref, b1_ref, b2_ref,
               res_ref, o_ref):
    x = x_ref[...]
    a = a0_ref[0] + a1_ref[0]
    h = jnp.maximum(_dot(x, u_ref[...]) + _dot(a, v_ref[...]) + b1_ref[...], 0.0)
    o_ref[...] = _dot(h, w2_ref[...]) + b2_ref[...] + res_ref[...]


def _node_mlp(x, agg, U, V, W2, b1, b2, res, bm):
    m, n = x.shape
    spec = pl.BlockSpec((bm, n), lambda i: (i, 0))
    wspec = pl.BlockSpec((n, n), lambda i: (0, 0))
    bspec = pl.BlockSpec((1, n), lambda i: (0, 0))
    aspec = pl.BlockSpec((1, bm, n), lambda i: (0, i, 0))
    return pl.pallas_call(
        _node_body,
        grid=(m // bm,),
        in_specs=[spec, aspec, aspec, wspec, wspec, wspec, bspec, bspec, spec],
        out_specs=spec,
        out_shape=jax.ShapeDtypeStruct((m, n), F32),
    )(x, agg[0:1], agg[1:2], U, V, W2, b1.reshape(1, n), b2.reshape(1, n), res)


def _policy_body(x_ref, w_ref, b_ref, o_ref):
    z = _dot(x_ref[...], w_ref[...]) + b_ref[...]
    o_ref[...] = jnp.clip(jax.nn.sigmoid(z), 0.0, 1.0)


def _policy(x, w, b, bm):
    m, k = x.shape
    return pl.pallas_call(
        _policy_body,
        grid=(m // bm,),
        in_specs=[
            pl.BlockSpec((bm, k), lambda i: (i, 0)),
            pl.BlockSpec((k, 1), lambda i: (0, 0)),
            pl.BlockSpec((1, 1), lambda i: (0, 0)),
        ],
        out_specs=pl.BlockSpec((bm, 1), lambda i: (i, 0)),
        out_shape=jax.ShapeDtypeStruct((m, 1), F32),
    )(x, w, b.reshape(1, 1))


def _sc_mesh():
    return plsc.VectorSubcoreMesh(core_axis_name="c", subcore_axis_name="s",
                                  num_cores=_NC, num_subcores=_NS)


def _sc_gather_call(P, Q, si2, di2):
    n = P.shape[1]
    n_ch = si2.shape[0]
    q = n_ch // _NW
    n_groups, n_tail = divmod(q, 3)

    @functools.partial(
        pl.kernel,
        out_type=jax.ShapeDtypeStruct((n_ch * _CH, n), F32),
        mesh=_sc_mesh(),
        scratch_types=[
            pltpu.VMEM((q, _CH), jnp.int32),
            pltpu.VMEM((q, _CH), jnp.int32),
            [pltpu.VMEM((_CH, n), F32) for _ in range(3)],
            [pltpu.VMEM((_CH, n), F32) for _ in range(3)],
            [pltpu.SemaphoreType.DMA for _ in range(3)],
            [pltpu.SemaphoreType.DMA for _ in range(3)],
        ],
    )
    def k(p_hbm, q_hbm, si_hbm, di_hbm, out_hbm, si_v, di_v, pg, qg, sg, sw):
        wid = lax.axis_index("s") * _NC + lax.axis_index("c")
        start = wid * q
        pltpu.sync_copy(si_hbm.at[pl.ds(start, q)], si_v)
        pltpu.sync_copy(di_hbm.at[pl.ds(start, q)], di_v)

        def add_rows(pgb, qgb):
            def add4(i, carry):
                base = i * 4
                for rr in range(4):
                    for t in range(n // _LANES):
                        s = pl.ds(t * _LANES, _LANES)
                        pgb[base + rr, s] = pgb[base + rr, s] + qgb[base + rr, s]
                return carry

            lax.fori_loop(0, _CH // 4, add4, 0)

        def out_slice(j):
            return out_hbm.at[pl.ds((start + j) * _CH, _CH)]

        def issue(j, s):
            cp = pltpu.async_copy(p_hbm.at[si_v.at[j]], pg[s], sg[s])
            cq = pltpu.async_copy(q_hbm.at[di_v.at[j]], qg[s], sg[s])
            return cp, cq

        def finish(j, s):
            pltpu.make_async_copy(p_hbm.at[si_v.at[j]], pg[s], sg[s]).wait()
            pltpu.make_async_copy(q_hbm.at[di_v.at[j]], qg[s], sg[s]).wait()
            add_rows(pg[s], qg[s])
            pltpu.async_copy(pg[s], out_slice(j), sw[s])

        def group(g, carry):
            for s in range(3):
                j = g * 3 + s

                @pl.when(g > 0)
                def _(s=s, j=j):
                    pltpu.make_async_copy(pg[s], out_slice(j - 3), sw[s]).wait()

                issue(j, s)
            for s in range(3):
                finish(g * 3 + s, s)
            return carry

        lax.fori_loop(0, n_groups, group, 0)

        last_write = {s: n_groups * 3 - 3 + s for s in range(3)}
        for t in range(n_tail):
            j = n_groups * 3 + t
            pltpu.make_async_copy(pg[t], out_slice(last_write[t]), sw[t]).wait()
            issue(j, t)
            finish(j, t)
            last_write[t] = j
        for s in range(3):
            pltpu.make_async_copy(pg[s], out_slice(last_write[s]), sw[s]).wait()

    return k(P, Q, si2, di2)


def _sc_scatter_call(e_feat, di2, n_nodes):
    n = e_feat.shape[1]
    n_ch = di2.shape[0]
    q = n_ch // _NW
    npairs = q // 2
    zrows = 128
    rows_per_tile = -(-n_nodes // (_NS * zrows)) * zrows
    n_pad = rows_per_tile * _NS

    @functools.partial(
        pl.kernel,
        out_type=jax.ShapeDtypeStruct((_NC, n_pad, n), F32),
        mesh=_sc_mesh(),
        scratch_types=[
            pltpu.VMEM((q, _CH), jnp.int32),
            pltpu.VMEM((_CH, n), F32),
            pltpu.VMEM((_CH, n), F32),
            pltpu.VMEM_SHARED((n_pad, n), F32),
            pltpu.SemaphoreType.DMA,
            pltpu.SemaphoreType.DMA,
            pltpu.SemaphoreType.DMA,
            pltpu.SemaphoreType.DMA,
        ],
    )
    def k(e_hbm, di_hbm, out_hbm, idx_v, rows0, rows1, agg_sh,
          sl0, sl1, ss0, ss1):
        cid = lax.axis_index("c")
        sid = lax.axis_index("s")
        wid = sid * _NC + cid
        start = wid * q

        def zrow(i, carry):
            for t in range(n // _LANES):
                rows0[i, pl.ds(t * _LANES, _LANES)] = jnp.zeros((_LANES,), F32)
            return carry

        lax.fori_loop(0, zrows, zrow, 0)
        for t in range(rows_per_tile // zrows):
            pltpu.sync_copy(
                rows0,
                agg_sh.at[pl.ds(sid * rows_per_tile + t * zrows, zrows)])
        pltpu.sync_copy(di_hbm.at[pl.ds(start, q)], idx_v)
        plsc.subcore_barrier()

        def e_slice(j):
            return e_hbm.at[pl.ds((start + j) * _CH, _CH)]

        def pair(j2, carry):
            j0 = j2 * 2
            j1 = j0 + 1

            @pl.when(j2 > 0)
            def _():
                pltpu.make_async_copy(
                    rows0, agg_sh.at[idx_v.at[j0 - 2]], ss0).wait()

            l0 = pltpu.async_copy(e_slice(j0), rows0, sl0)

            @pl.when(j2 > 0)
            def _():
                pltpu.make_async_copy(
                    rows1, agg_sh.at[idx_v.at[j1 - 2]], ss1).wait()

            l1 = pltpu.async_copy(e_slice(j1), rows1, sl1)
            l0.wait()
            pltpu.async_copy(rows0, agg_sh.at[idx_v.at[j0]], ss0, add=True)
            l1.wait()
            pltpu.async_copy(rows1, agg_sh.at[idx_v.at[j1]], ss1, add=True)
            return carry

        lax.fori_loop(0, npairs, pair, 0)
        pltpu.make_async_copy(
            rows0, agg_sh.at[idx_v.at[q - 2]], ss0).wait()
        pltpu.make_async_copy(
            rows1, agg_sh.at[idx_v.at[q - 1]], ss1).wait()
        plsc.subcore_barrier()

        for t in range(rows_per_tile // zrows):
            sl = pl.ds(sid * rows_per_tile + t * zrows, zrows)
            pltpu.sync_copy(agg_sh.at[sl], rows0)
            pltpu.sync_copy(rows0, out_hbm.at[cid, sl])

    return k(e_feat, di2)


def kernel(nf, ef, edge_index, W_ne, b_ne, W_ee, b_ee, eW1, eb1, eW2, eb2,
           nW1, nb1, nW2, nb2, pW, pb):
    n_nodes, latent = nf.shape[0], W_ne.shape[1]
    n_layers = eW1.shape[0]
    bn = 2000
    be = 2560

    sidx = edge_index[0].astype(jnp.int32)
    didx = edge_index[1].astype(jnp.int32)

    e_total = sidx.shape[0]
    n_ch = -(-(e_total // _CH) // (2 * _NW)) * (2 * _NW)
    pad = n_ch * _CH - e_total
    pad_idx = jnp.arange(pad, dtype=jnp.int32) % n_nodes
    si2 = jnp.concatenate([sidx, pad_idx]).reshape(n_ch, _CH)
    di2 = jnp.concatenate([didx, pad_idx]).reshape(n_ch, _CH)

    unf0 = _mm_bias(nf, W_ne, b_ne, bn)
    uef0 = _mm_bias(ef, W_ee, b_ee, be)

    unf = unf0
    e_prev = None
    for l in range(n_layers):
        A = eW1[l, :latent, :]
        B = eW1[l, latent:2 * latent, :]
        C = eW1[l, 2 * latent:, :]
        P, Q = _pq_project(unf, A, B, bn)
        G = _sc_gather_call(P, Q, si2, di2)
        nreal = e_total // be
        if e_prev is None:
            e_new = _edge_mlp(uef0, None, G, C, eW2[l], eb1[l], eb2[l], be,
                              nreal)
        else:
            e_new = _edge_mlp(e_prev, uef0, G, C, eW2[l], eb1[l], eb2[l], be,
                              nreal)
        agg = _sc_scatter_call(e_new, di2, n_nodes)[:, :n_nodes, :]
        U = nW1[l, :latent, :]
        V = nW1[l, latent:, :]
        unf = _node_mlp(unf, agg, U, V, nW2[l], nb1[l], nb2[l], unf0, bn)
        e_prev = e_new

    return _policy(unf, pW, pb, bn)


# speedup vs baseline: 4.2639x; 1.0779x over previous
import functools

import jax
import jax.numpy as jnp
from jax import lax
from jax.experimental import pallas as pl
from jax.experimental.pallas import tpu as pltpu
from jax.experimental.pallas import tpu_sc as plsc

F32 = jnp.float32
_NC, _NS = 2, 16
_NW = _NC * _NS
_LANES = 16
_CH = 128

def _dot(a, b):
    return jnp.dot(a.astype(jnp.bfloat16), b.astype(jnp.bfloat16),
                   preferred_element_type=F32)


def _mm_bias_body(x_ref, w_ref, b_ref, o_ref):
    o_ref[...] = _dot(x_ref[...], w_ref[...]) + b_ref[...]


def _mm_bias(x, w, b, bm):
    m, k = x.shape
    n = w.shape[1]
    return pl.pallas_call(
        _mm_bias_body,
        grid=(m // bm,),
        in_specs=[
            pl.BlockSpec((bm, k), lambda i: (i, 0)),
            pl.BlockSpec((k, n), lambda i: (0, 0)),
            pl.BlockSpec((1, n), lambda i: (0, 0)),
        ],
        out_specs=pl.BlockSpec((bm, n), lambda i: (i, 0)),
        out_shape=jax.ShapeDtypeStruct((m, n), F32),
    )(x, w, b.reshape(1, n))


def _pq_body(x_ref, a_ref, b_ref, p_ref, q_ref):
    x = x_ref[...]
    p_ref[...] = _dot(x, a_ref[...])
    q_ref[...] = _dot(x, b_ref[...])


def _pq_project(unf, A, B, bm):
    m, k = unf.shape
    n = A.shape[1]
    return pl.pallas_call(
        _pq_body,
        grid=(m // bm,),
        in_specs=[
            pl.BlockSpec((bm, k), lambda i: (i, 0)),
            pl.BlockSpec((k, n), lambda i: (0, 0)),
            pl.BlockSpec((k, n), lambda i: (0, 0)),
        ],
        out_specs=[
            pl.BlockSpec((bm, n), lambda i: (i, 0)),
            pl.BlockSpec((bm, n), lambda i: (i, 0)),
        ],
        out_shape=[
            jax.ShapeDtypeStruct((m, n), F32),
            jax.ShapeDtypeStruct((m, n), F32),
        ],
    )(unf, A, B)


def _edge0_body(u_ref, g_ref, c_ref, w2_ref, b1_ref, b2_ref, o_ref, *, nreal):
    @pl.when(pl.program_id(0) < nreal)
    def _():
        h = jnp.maximum(
            _dot(u_ref[...], c_ref[...]) + g_ref[...] + b1_ref[...], 0.0)
        o_ref[...] = _dot(h, w2_ref[...]) + b2_ref[...]

    @pl.when(pl.program_id(0) >= nreal)
    def _():
        o_ref[...] = jnp.zeros_like(o_ref)


def _edge_res_body(u_ref, r_ref, g_ref, c_ref, w2_ref, b1_ref, b2_ref, o_ref,
                   *, nreal):
    @pl.when(pl.program_id(0) < nreal)
    def _():
        u = u_ref[...] + r_ref[...]
        h = jnp.maximum(_dot(u, c_ref[...]) + g_ref[...] + b1_ref[...], 0.0)
        o_ref[...] = _dot(h, w2_ref[...]) + b2_ref[...]

    @pl.when(pl.program_id(0) >= nreal)
    def _():
        o_ref[...] = jnp.zeros_like(o_ref)


def _edge_mlp(u, res, g, C, W2, b1, b2, bm, nreal, off=0):
    n = u.shape[1]
    m_pad = g.shape[0]

    def spec(arr):
        nb = arr.shape[0] // bm
        if nb == m_pad // bm and off == 0:
            return pl.BlockSpec((bm, n), lambda i: (i, 0))
        if arr.shape[0] == m_pad:
            return pl.BlockSpec((bm, n), lambda i: (i, 0))
        return pl.BlockSpec(
            (bm, n), lambda i, nb=nb: (jnp.minimum(i + off, nb - 1), 0))

    wspec = pl.BlockSpec((n, n), lambda i: (0, 0))
    bspec = pl.BlockSpec((1, n), lambda i: (0, 0))
    if res is None:
        body = functools.partial(_edge0_body, nreal=nreal)
        args = (u, g, C, W2, b1.reshape(1, n), b2.reshape(1, n))
        in_specs = [spec(u), spec(g), wspec, wspec, bspec, bspec]
    else:
        body = functools.partial(_edge_res_body, nreal=nreal)
        args = (u, res, g, C, W2, b1.reshape(1, n), b2.reshape(1, n))
        in_specs = [spec(u), spec(res), spec(g), wspec, wspec, bspec, bspec]
    return pl.pallas_call(
        body,
        grid=(m_pad // bm,),
        in_specs=in_specs,
        out_specs=spec(g),
        out_shape=jax.ShapeDtypeStruct((m_pad, n), F32),
    )(*args)


def _node_body(x_ref, a0_ref, a1_ref, a2_ref, a3_ref, u_ref, v_ref, w2_ref,
               b1_ref, b2_ref, res_ref, o_ref):
    x = x_ref[...]
    a = (a0_ref[0] + a1_ref[0]) + (a2_ref[0] + a3_ref[0])
    h = jnp.maximum(_dot(x, u_ref[...]) + _dot(a, v_ref[...]) + b1_ref[...], 0.0)
    o_ref[...] = _dot(h, w2_ref[...]) + b2_ref[...] + res_ref[...]


def _node_mlp(x, agg_a, agg_b, U, V, W2, b1, b2, res, bm):
    m, n = x.shape
    spec = pl.BlockSpec((bm, n), lambda i: (i, 0))
    wspec = pl.BlockSpec((n, n), lambda i: (0, 0))
    bspec = pl.BlockSpec((1, n), lambda i: (0, 0))
    aspec = pl.BlockSpec((1, bm, n), lambda i: (0, i, 0))
    return pl.pallas_call(
        _node_body,
        grid=(m // bm,),
        in_specs=[spec, aspec, aspec, aspec, aspec, wspec, wspec, wspec,
                  bspec, bspec, spec],
        out_specs=spec,
        out_shape=jax.ShapeDtypeStruct((m, n), F32),
    )(x, agg_a[0:1], agg_a[1:2], agg_b[0:1], agg_b[1:2], U, V, W2,
      b1.reshape(1, n), b2.reshape(1, n), res)


def _policy_body(x_ref, w_ref, b_ref, o_ref):
    z = _dot(x_ref[...], w_ref[...]) + b_ref[...]
    o_ref[...] = jnp.clip(jax.nn.sigmoid(z), 0.0, 1.0)


def _policy(x, w, b, bm):
    m, k = x.shape
    return pl.pallas_call(
        _policy_body,
        grid=(m // bm,),
        in_specs=[
            pl.BlockSpec((bm, k), lambda i: (i, 0)),
            pl.BlockSpec((k, 1), lambda i: (0, 0)),
            pl.BlockSpec((1, 1), lambda i: (0, 0)),
        ],
        out_specs=pl.BlockSpec((bm, 1), lambda i: (i, 0)),
        out_shape=jax.ShapeDtypeStruct((m, 1), F32),
    )(x, w, b.reshape(1, 1))


def _sc_mesh():
    return plsc.VectorSubcoreMesh(core_axis_name="c", subcore_axis_name="s",
                                  num_cores=_NC, num_subcores=_NS)


def _sc_gather_call(P, Q, si2, di2):
    n = P.shape[1]
    n_ch = si2.shape[0]
    q = n_ch // _NW
    n_groups, n_tail = divmod(q, 3)

    @functools.partial(
        pl.kernel,
        out_type=jax.ShapeDtypeStruct((n_ch * _CH, n), F32),
        mesh=_sc_mesh(),
        scratch_types=[
            pltpu.VMEM((q, _CH), jnp.int32),
            pltpu.VMEM((q, _CH), jnp.int32),
            [pltpu.VMEM((_CH, n), F32) for _ in range(3)],
            [pltpu.VMEM((_CH, n), F32) for _ in range(3)],
            [pltpu.SemaphoreType.DMA for _ in range(3)],
            [pltpu.SemaphoreType.DMA for _ in range(3)],
        ],
    )
    def k(p_hbm, q_hbm, si_hbm, di_hbm, out_hbm, si_v, di_v, pg, qg, sg, sw):
        wid = lax.axis_index("s") * _NC + lax.axis_index("c")
        start = wid * q
        pltpu.sync_copy(si_hbm.at[pl.ds(start, q)], si_v)
        pltpu.sync_copy(di_hbm.at[pl.ds(start, q)], di_v)

        def add_rows(pgb, qgb):
            def add4(i, carry):
                base = i * 4
                for rr in range(4):
                    for t in range(n // _LANES):
                        s = pl.ds(t * _LANES, _LANES)
                        pgb[base + rr, s] = pgb[base + rr, s] + qgb[base + rr, s]
                return carry

            lax.fori_loop(0, _CH // 4, add4, 0)

        def out_slice(j):
            return out_hbm.at[pl.ds((start + j) * _CH, _CH)]

        def issue(j, s):
            cp = pltpu.async_copy(p_hbm.at[si_v.at[j]], pg[s], sg[s])
            cq = pltpu.async_copy(q_hbm.at[di_v.at[j]], qg[s], sg[s])
            return cp, cq

        def finish(j, s):
            pltpu.make_async_copy(p_hbm.at[si_v.at[j]], pg[s], sg[s]).wait()
            pltpu.make_async_copy(q_hbm.at[di_v.at[j]], qg[s], sg[s]).wait()
            add_rows(pg[s], qg[s])
            pltpu.async_copy(pg[s], out_slice(j), sw[s])

        def group(g, carry):
            for s in range(3):
                j = g * 3 + s

                @pl.when(g > 0)
                def _(s=s, j=j):
                    pltpu.make_async_copy(pg[s], out_slice(j - 3), sw[s]).wait()

                issue(j, s)
            for s in range(3):
                finish(g * 3 + s, s)
            return carry

        lax.fori_loop(0, n_groups, group, 0)

        last_write = {s: n_groups * 3 - 3 + s for s in range(3)}
        for t in range(n_tail):
            j = n_groups * 3 + t
            pltpu.make_async_copy(pg[t], out_slice(last_write[t]), sw[t]).wait()
            issue(j, t)
            finish(j, t)
            last_write[t] = j
        for s in range(3):
            pltpu.make_async_copy(pg[s], out_slice(last_write[s]), sw[s]).wait()

    return k(P, Q, si2, di2)


def _sc_scatter_call(e_feat, di2, n_nodes):
    n = e_feat.shape[1]
    n_ch = di2.shape[0]
    q = n_ch // _NW
    npairs = q // 2
    zrows = 128
    rows_per_tile = -(-n_nodes // (_NS * zrows)) * zrows
    n_pad = rows_per_tile * _NS

    @functools.partial(
        pl.kernel,
        out_type=jax.ShapeDtypeStruct((_NC, n_pad, n), F32),
        mesh=_sc_mesh(),
        scratch_types=[
            pltpu.VMEM((q, _CH), jnp.int32),
            pltpu.VMEM((_CH, n), F32),
            pltpu.VMEM((_CH, n), F32),
            pltpu.VMEM_SHARED((n_pad, n), F32),
            pltpu.SemaphoreType.DMA,
            pltpu.SemaphoreType.DMA,
            pltpu.SemaphoreType.DMA,
            pltpu.SemaphoreType.DMA,
        ],
    )
    def k(e_hbm, di_hbm, out_hbm, idx_v, rows0, rows1, agg_sh,
          sl0, sl1, ss0, ss1):
        cid = lax.axis_index("c")
        sid = lax.axis_index("s")
        wid = sid * _NC + cid
        start = wid * q

        def zrow(i, carry):
            for t in range(n // _LANES):
                rows0[i, pl.ds(t * _LANES, _LANES)] = jnp.zeros((_LANES,), F32)
            return carry

        lax.fori_loop(0, zrows, zrow, 0)
        for t in range(rows_per_tile // zrows):
            pltpu.sync_copy(
                rows0,
                agg_sh.at[pl.ds(sid * rows_per_tile + t * zrows, zrows)])
        pltpu.sync_copy(di_hbm.at[pl.ds(start, q)], idx_v)
        plsc.subcore_barrier()

        def e_slice(j):
            return e_hbm.at[pl.ds((start + j) * _CH, _CH)]

        def pair(j2, carry):
            j0 = j2 * 2
            j1 = j0 + 1

            @pl.when(j2 > 0)
            def _():
                pltpu.make_async_copy(
                    rows0, agg_sh.at[idx_v.at[j0 - 2]], ss0).wait()

            l0 = pltpu.async_copy(e_slice(j0), rows0, sl0)

            @pl.when(j2 > 0)
            def _():
                pltpu.make_async_copy(
                    rows1, agg_sh.at[idx_v.at[j1 - 2]], ss1).wait()

            l1 = pltpu.async_copy(e_slice(j1), rows1, sl1)
            l0.wait()
            pltpu.async_copy(rows0, agg_sh.at[idx_v.at[j0]], ss0, add=True)
            l1.wait()
            pltpu.async_copy(rows1, agg_sh.at[idx_v.at[j1]], ss1, add=True)
            return carry

        lax.fori_loop(0, npairs, pair, 0)
        pltpu.make_async_copy(
            rows0, agg_sh.at[idx_v.at[q - 2]], ss0).wait()
        pltpu.make_async_copy(
            rows1, agg_sh.at[idx_v.at[q - 1]], ss1).wait()
        plsc.subcore_barrier()

        for t in range(rows_per_tile // zrows):
            sl = pl.ds(sid * rows_per_tile + t * zrows, zrows)
            pltpu.sync_copy(agg_sh.at[sl], rows0)
            pltpu.sync_copy(rows0, out_hbm.at[cid, sl])

    return k(e_feat, di2)


def kernel(nf, ef, edge_index, W_ne, b_ne, W_ee, b_ee, eW1, eb1, eW2, eb2,
           nW1, nb1, nW2, nb2, pW, pb):
    n_nodes, latent = nf.shape[0], W_ne.shape[1]
    n_layers = eW1.shape[0]
    bn = 2000
    be = 2560

    sidx = edge_index[0].astype(jnp.int32)
    didx = edge_index[1].astype(jnp.int32)

    e_total = sidx.shape[0]
    n_ch = -(-(e_total // _CH) // (4 * _NW)) * (4 * _NW)
    pad = n_ch * _CH - e_total
    pad_idx = jnp.arange(pad, dtype=jnp.int32) % n_nodes
    si2 = jnp.concatenate([sidx, pad_idx]).reshape(n_ch, _CH)
    di2 = jnp.concatenate([didx, pad_idx]).reshape(n_ch, _CH)
    half = n_ch // 2
    si2_h = (si2[:half], si2[half:])
    di2_h = (di2[:half], di2[half:])
    e_half = half * _CH
    blk_off = (0, e_half // be)
    nreal_h = (min(e_total, e_half) // be,
               max(e_total - e_half, 0) // be)

    unf0 = _mm_bias(nf, W_ne, b_ne, bn)
    uef0 = _mm_bias(ef, W_ee, b_ee, be)

    unf = unf0
    e_prev = (None, None)
    for l in range(n_layers):
        A = eW1[l, :latent, :]
        B = eW1[l, latent:2 * latent, :]
        C = eW1[l, 2 * latent:, :]
        P, Q = _pq_project(unf, A, B, bn)
        G = [_sc_gather_call(P, Q, si2_h[h], di2_h[h]) for h in range(2)]
        e_new = []
        agg = []
        for h in range(2):
            e_new.append(_edge_mlp(
                e_prev[h] if e_prev[h] is not None else uef0,
                uef0 if e_prev[h] is not None else None,
                G[h], C, eW2[l], eb1[l], eb2[l], be, nreal_h[h],
                off=blk_off[h]))
            agg.append(_sc_scatter_call(e_new[h], di2_h[h], n_nodes)
                       [:, :n_nodes, :])
        U = nW1[l, :latent, :]
        V = nW1[l, latent:, :]
        unf = _node_mlp(unf, agg[0], agg[1], U, V, nW2[l], nb1[l], nb2[l],
                        unf0, bn)
        e_prev = tuple(e_new)

    return _policy(unf, pW, pb, bn)
